# Initial kernel scaffold; baseline (speedup 1.0000x reference)
import jax
import jax.numpy as jnp
from jax.experimental import pallas as pl


def kernel(user_ids, item_ids, user_table, item_table, user_bias_table, item_bias_table, global_bias):
    raise NotImplementedError("write your pallas kernel here")


# speedup vs baseline: 1.3732x; 1.3732x over previous
import functools

import jax
import jax.numpy as jnp
from jax import lax
from jax.experimental import pallas as pl
from jax.experimental.pallas import tpu as pltpu
from jax.experimental.pallas import tpu_sc as plsc

NC = 2
NS = 16
LANES = 16


def kernel(user_ids, item_ids, user_table, item_table, user_bias_table,
           item_bias_table, global_bias):
    B = user_ids.shape[0]
    D = user_table.shape[1]
    NW = NC * NS
    b_per_w = B // NW
    CH = 128
    n_ch = b_per_w // CH
    n_d = D // LANES

    mesh = plsc.VectorSubcoreMesh(core_axis_name="c", subcore_axis_name="s",
                                  num_cores=NC, num_subcores=NS)

    @functools.partial(
        pl.kernel,
        out_type=jax.ShapeDtypeStruct((B,), jnp.float32),
        mesh=mesh,
        compiler_params=pltpu.CompilerParams(needs_layout_passes=False),
        scratch_types=[
            pltpu.VMEM((CH,), jnp.int32),
            pltpu.VMEM((CH,), jnp.int32),
            pltpu.VMEM((CH, D), jnp.float32),
            pltpu.VMEM((CH, D), jnp.float32),
            pltpu.VMEM((CH,), jnp.float32),
            pltpu.VMEM((CH,), jnp.float32),
            pltpu.VMEM((b_per_w,), jnp.float32),
            pltpu.VMEM((LANES,), jnp.float32),
            pltpu.VMEM((LANES * LANES,), jnp.float32),
            pltpu.SemaphoreType.DMA,
            pltpu.SemaphoreType.DMA,
            pltpu.SemaphoreType.DMA,
            pltpu.SemaphoreType.DMA,
        ],
    )
    def mf_kernel(uids_hbm, iids_hbm, ut_hbm, it_hbm, ubt_hbm, ibt_hbm,
                  gb_hbm, out_hbm, uidx, iidx, urows, irows, ub, ib,
                  outb, gbv, part, sem_u, sem_i, sem_ub, sem_ib):
        wid = lax.axis_index("s") * NC + lax.axis_index("c")
        base = wid * b_per_w

        pltpu.sync_copy(gb_hbm, gbv.at[pl.ds(0, 1)])
        gbias = gbv[pl.ds(0, LANES)][0]

        lanes = lax.iota(jnp.int32, LANES)

        for c in range(n_ch):
            off = base + c * CH
            pltpu.sync_copy(uids_hbm.at[pl.ds(off, CH)], uidx)
            pltpu.sync_copy(iids_hbm.at[pl.ds(off, CH)], iidx)
            cu = pltpu.async_copy(ut_hbm.at[uidx], urows, sem_u)
            ci = pltpu.async_copy(it_hbm.at[iidx], irows, sem_i)
            cub = pltpu.async_copy(ubt_hbm.at[uidx], ub, sem_ub)
            cib = pltpu.async_copy(ibt_hbm.at[iidx], ib, sem_ib)
            cu.wait()
            ci.wait()
            cub.wait()
            cib.wait()

            def grp_body(g, carry):
                r0 = g * LANES
                for j in range(LANES):
                    r = r0 + j
                    acc = (urows[r, pl.ds(0, LANES)] *
                           irows[r, pl.ds(0, LANES)])
                    for d in range(1, n_d):
                        acc = acc + (urows[r, pl.ds(d * LANES, LANES)] *
                                     irows[r, pl.ds(d * LANES, LANES)])
                    part[pl.ds(j * LANES, LANES)] = acc
                rowsel = lanes * LANES
                dots = plsc.load_gather(part, [rowsel])
                for l in range(1, LANES):
                    dots = dots + plsc.load_gather(part, [rowsel + l])
                dots = (dots + ub[pl.ds(r0, LANES)] + ib[pl.ds(r0, LANES)]
                        + gbias)
                outb[pl.ds(c * CH + r0, LANES)] = dots
                return carry

            lax.fori_loop(0, CH // LANES, grp_body, 0)

        pltpu.sync_copy(outb, out_hbm.at[pl.ds(base, b_per_w)])

    out = mf_kernel(user_ids, item_ids, user_table, item_table,
                    user_bias_table.reshape(-1), item_bias_table.reshape(-1),
                    global_bias)
    return out.reshape(B, 1)


# speedup vs baseline: 1.6358x; 1.1912x over previous
import functools

import jax
import jax.numpy as jnp
from jax import lax
from jax.experimental import pallas as pl
from jax.experimental.pallas import tpu as pltpu
from jax.experimental.pallas import tpu_sc as plsc

NC = 2
NS = 16
LANES = 16


def kernel(user_ids, item_ids, user_table, item_table, user_bias_table,
           item_bias_table, global_bias):
    B = user_ids.shape[0]
    D = user_table.shape[1]
    NW = NC * NS
    b_per_w = B // NW
    CH = 128
    n_ch = b_per_w // CH
    n_d = D // LANES

    mesh = plsc.VectorSubcoreMesh(core_axis_name="c", subcore_axis_name="s",
                                  num_cores=NC, num_subcores=NS)

    @functools.partial(
        pl.kernel,
        out_type=jax.ShapeDtypeStruct((B,), jnp.float32),
        mesh=mesh,
        compiler_params=pltpu.CompilerParams(needs_layout_passes=False),
        scratch_types=[
            pltpu.VMEM((2, CH), jnp.int32),
            pltpu.VMEM((2, CH), jnp.int32),
            pltpu.VMEM((2, CH, D), jnp.float32),
            pltpu.VMEM((2, CH, D), jnp.float32),
            pltpu.VMEM((2, CH), jnp.float32),
            pltpu.VMEM((2, CH), jnp.float32),
            pltpu.VMEM((b_per_w,), jnp.float32),
            pltpu.VMEM((LANES,), jnp.float32),
            pltpu.VMEM((LANES * LANES,), jnp.float32),
            pltpu.SemaphoreType.DMA,
            pltpu.SemaphoreType.DMA,
        ],
    )
    def mf_kernel(uids_hbm, iids_hbm, ut_hbm, it_hbm, ubt_hbm, ibt_hbm,
                  gb_hbm, out_hbm, uidx, iidx, urows, irows, ub, ib,
                  outb, gbv, part, sem0, sem1):
        wid = lax.axis_index("s") * NC + lax.axis_index("c")
        base = wid * b_per_w
        sems = (sem0, sem1)

        pltpu.sync_copy(gb_hbm, gbv.at[pl.ds(0, 1)])
        gbias = gbv[pl.ds(0, LANES)][0]

        lanes = lax.iota(jnp.int32, LANES)

        def start_chunk(c):
            s = c % 2
            off = base + c * CH
            pltpu.sync_copy(uids_hbm.at[pl.ds(off, CH)], uidx.at[s])
            pltpu.sync_copy(iids_hbm.at[pl.ds(off, CH)], iidx.at[s])
            sem = sems[s]
            return (
                pltpu.async_copy(ut_hbm.at[uidx.at[s]], urows.at[s], sem),
                pltpu.async_copy(it_hbm.at[iidx.at[s]], irows.at[s], sem),
                pltpu.async_copy(ubt_hbm.at[uidx.at[s]], ub.at[s], sem),
                pltpu.async_copy(ibt_hbm.at[iidx.at[s]], ib.at[s], sem),
            )

        def compute_chunk(c):
            s = c % 2
            u2 = urows.at[s]
            i2 = irows.at[s]

            def grp_body(g, carry):
                r0 = g * LANES
                for j in range(LANES):
                    r = r0 + j
                    acc = (u2[r, pl.ds(0, LANES)] *
                           i2[r, pl.ds(0, LANES)])
                    for d in range(1, n_d):
                        acc = acc + (u2[r, pl.ds(d * LANES, LANES)] *
                                     i2[r, pl.ds(d * LANES, LANES)])
                    part[pl.ds(j * LANES, LANES)] = acc
                rowsel = lanes * LANES
                dots = plsc.load_gather(part, [rowsel])
                for l in range(1, LANES):
                    dots = dots + plsc.load_gather(part, [rowsel + l])
                dots = (dots + ub[s, pl.ds(r0, LANES)]
                        + ib[s, pl.ds(r0, LANES)] + gbias)
                outb[pl.ds(c * CH + r0, LANES)] = dots
                return carry

            lax.fori_loop(0, CH // LANES, grp_body, 0)

        inflight = start_chunk(0)
        for c in range(n_ch):
            if c + 1 < n_ch:
                nxt = start_chunk(c + 1)
            else:
                nxt = None
            for cp in inflight:
                cp.wait()
            compute_chunk(c)
            inflight = nxt

        pltpu.sync_copy(outb, out_hbm.at[pl.ds(base, b_per_w)])

    out = mf_kernel(user_ids, item_ids, user_table, item_table,
                    user_bias_table.reshape(-1), item_bias_table.reshape(-1),
                    global_bias)
    return out.reshape(B, 1)


# speedup vs baseline: 1.6721x; 1.0222x over previous
import functools

import jax
import jax.numpy as jnp
from jax import lax
from jax.experimental import pallas as pl
from jax.experimental.pallas import tpu as pltpu
from jax.experimental.pallas import tpu_sc as plsc

NC = 2
NS = 16
LANES = 16


def kernel(user_ids, item_ids, user_table, item_table, user_bias_table,
           item_bias_table, global_bias):
    B = user_ids.shape[0]
    D = user_table.shape[1]
    NW = NC * NS
    b_per_w = B // NW
    CH = 128
    n_ch = b_per_w // CH
    n_d = D // LANES

    mesh = plsc.VectorSubcoreMesh(core_axis_name="c", subcore_axis_name="s",
                                  num_cores=NC, num_subcores=NS)

    @functools.partial(
        pl.kernel,
        out_type=jax.ShapeDtypeStruct((B,), jnp.float32),
        mesh=mesh,
        compiler_params=pltpu.CompilerParams(needs_layout_passes=False),
        scratch_types=[
            pltpu.VMEM((b_per_w,), jnp.int32),
            pltpu.VMEM((b_per_w,), jnp.int32),
            pltpu.VMEM((2, CH, D), jnp.float32),
            pltpu.VMEM((2, CH, D), jnp.float32),
            pltpu.VMEM((b_per_w,), jnp.float32),
            pltpu.VMEM((b_per_w,), jnp.float32),
            pltpu.VMEM((b_per_w,), jnp.float32),
            pltpu.VMEM((LANES,), jnp.float32),
            pltpu.VMEM((LANES * LANES,), jnp.float32),
            pltpu.SemaphoreType.DMA,
            pltpu.SemaphoreType.DMA,
            pltpu.SemaphoreType.DMA,
        ],
    )
    def mf_kernel(uids_hbm, iids_hbm, ut_hbm, it_hbm, ubt_hbm, ibt_hbm,
                  gb_hbm, out_hbm, uidx, iidx, urows, irows, ub, ib,
                  outb, gbv, part, sem0, sem1, semb):
        wid = lax.axis_index("s") * NC + lax.axis_index("c")
        base = wid * b_per_w
        sems = (sem0, sem1)

        pltpu.sync_copy(gb_hbm, gbv.at[pl.ds(0, 1)])
        pltpu.sync_copy(uids_hbm.at[pl.ds(base, b_per_w)], uidx)
        pltpu.sync_copy(iids_hbm.at[pl.ds(base, b_per_w)], iidx)
        gbias = gbv[pl.ds(0, LANES)][0]

        lanes = lax.iota(jnp.int32, LANES)

        def fire_rows(c):
            s = c % 2
            ui = uidx.at[pl.ds(c * CH, CH)]
            ii = iidx.at[pl.ds(c * CH, CH)]
            return (
                pltpu.async_copy(ut_hbm.at[ui], urows.at[s], sems[s]),
                pltpu.async_copy(it_hbm.at[ii], irows.at[s], sems[s]),
            )

        def fire_bias(c):
            ui = uidx.at[pl.ds(c * CH, CH)]
            ii = iidx.at[pl.ds(c * CH, CH)]
            dst_u = ub.at[pl.ds(c * CH, CH)]
            dst_i = ib.at[pl.ds(c * CH, CH)]
            return (
                pltpu.async_copy(ubt_hbm.at[ui], dst_u, semb),
                pltpu.async_copy(ibt_hbm.at[ii], dst_i, semb),
            )

        pending = {0: fire_rows(0), 1: fire_rows(1)}
        bias_cps = []
        for c in range(n_ch):
            bias_cps.extend(fire_bias(c))
            if 1 <= c and c + 1 < n_ch:
                pending[c + 1] = fire_rows(c + 1)
            for cp in pending.pop(c):
                cp.wait()
            s = c % 2
            u2 = urows.at[s]
            i2 = irows.at[s]

            def grp_body(g, carry):
                r0 = g * LANES
                for j in range(LANES):
                    r = r0 + j
                    acc = (u2[r, pl.ds(0, LANES)] *
                           i2[r, pl.ds(0, LANES)])
                    for d in range(1, n_d):
                        acc = acc + (u2[r, pl.ds(d * LANES, LANES)] *
                                     i2[r, pl.ds(d * LANES, LANES)])
                    part[pl.ds(j * LANES, LANES)] = acc
                rowsel = lanes * LANES
                dots = plsc.load_gather(part, [rowsel])
                for l in range(1, LANES):
                    dots = dots + plsc.load_gather(part, [rowsel + l])
                outb[pl.ds(c * CH + r0, LANES)] = dots
                return carry

            lax.fori_loop(0, CH // LANES, grp_body, 0)

        for cp in bias_cps:
            cp.wait()

        def bias_body(g, carry):
            r0 = g * LANES
            sl = pl.ds(r0, LANES)
            outb[sl] = outb[sl] + ub[sl] + ib[sl] + gbias
            return carry

        lax.fori_loop(0, b_per_w // LANES, bias_body, 0)

        pltpu.sync_copy(outb, out_hbm.at[pl.ds(base, b_per_w)])

    out = mf_kernel(user_ids, item_ids, user_table, item_table,
                    user_bias_table.reshape(-1), item_bias_table.reshape(-1),
                    global_bias)
    return out.reshape(B, 1)


# speedup vs baseline: 1.8718x; 1.1195x over previous
import functools

import jax
import jax.numpy as jnp
from jax import lax
from jax.experimental import pallas as pl
from jax.experimental.pallas import tpu as pltpu
from jax.experimental.pallas import tpu_sc as plsc

NC = 2
NS = 16
LANES = 16


def kernel(user_ids, item_ids, user_table, item_table, user_bias_table,
           item_bias_table, global_bias):
    B = user_ids.shape[0]
    D = user_table.shape[1]
    NW = NC * NS
    b_per_w = B // NW
    CH = 128
    n_ch = b_per_w // CH
    n_d = D // LANES

    mesh = plsc.VectorSubcoreMesh(core_axis_name="c", subcore_axis_name="s",
                                  num_cores=NC, num_subcores=NS)

    @functools.partial(
        pl.kernel,
        out_type=jax.ShapeDtypeStruct((B,), jnp.float32),
        mesh=mesh,
        compiler_params=pltpu.CompilerParams(needs_layout_passes=False),
        scratch_types=[
            pltpu.VMEM((b_per_w,), jnp.int32),
            pltpu.VMEM((b_per_w,), jnp.int32),
            pltpu.VMEM((2, CH, D), jnp.float32),
            pltpu.VMEM((2, CH, D), jnp.float32),
            pltpu.VMEM((b_per_w,), jnp.float32),
            pltpu.VMEM((b_per_w,), jnp.float32),
            pltpu.VMEM((b_per_w,), jnp.float32),
            pltpu.VMEM((LANES,), jnp.float32),
            pltpu.VMEM((LANES * LANES,), jnp.float32),
            pltpu.SemaphoreType.DMA,
            pltpu.SemaphoreType.DMA,
            pltpu.SemaphoreType.DMA,
        ],
    )
    def mf_kernel(uids_hbm, iids_hbm, ut_hbm, it_hbm, ubt_hbm, ibt_hbm,
                  gb_hbm, out_hbm, uidx, iidx, urows, irows, ub, ib,
                  outb, gbv, part, sem0, sem1, semb):
        wid = lax.axis_index("s") * NC + lax.axis_index("c")
        base = wid * b_per_w
        sems = (sem0, sem1)

        cg = pltpu.async_copy(gb_hbm, gbv.at[pl.ds(0, 1)], semb)
        c0 = pltpu.async_copy(uids_hbm.at[pl.ds(base, b_per_w)], uidx, sem0)
        c1 = pltpu.async_copy(iids_hbm.at[pl.ds(base, b_per_w)], iidx, sem1)
        c0.wait()
        c1.wait()
        cg.wait()
        gbias = gbv[pl.ds(0, LANES)][0]

        lanes = lax.iota(jnp.int32, LANES)

        sched = [(c * CH, CH) for c in range(n_ch)]

        def fire_rows(k):
            s = k % 2
            off, sz = sched[k]
            ui = uidx.at[pl.ds(off, sz)]
            ii = iidx.at[pl.ds(off, sz)]
            du = urows.at[s].at[pl.ds(0, sz)]
            di = irows.at[s].at[pl.ds(0, sz)]
            return (
                pltpu.async_copy(ut_hbm.at[ui], du, sems[s]),
                pltpu.async_copy(it_hbm.at[ii], di, sems[s]),
            )

        def fire_bias(c):
            ui = uidx.at[pl.ds(c * CH, CH)]
            ii = iidx.at[pl.ds(c * CH, CH)]
            dst_u = ub.at[pl.ds(c * CH, CH)]
            dst_i = ib.at[pl.ds(c * CH, CH)]
            return (
                pltpu.async_copy(ubt_hbm.at[0].at[ui], dst_u, semb),
                pltpu.async_copy(ibt_hbm.at[0].at[ii], dst_i, semb),
            )

        n_k = len(sched)
        pending = {0: fire_rows(0), 1: fire_rows(1)}
        bias_cps = []
        for k in range(n_k):
            if k < n_ch:
                bias_cps.extend(fire_bias(k))
            if 1 <= k and k + 1 < n_k:
                pending[k + 1] = fire_rows(k + 1)
            for cp in pending.pop(k):
                cp.wait()
            s = k % 2
            off, sz = sched[k]
            u2 = urows.at[s]
            i2 = irows.at[s]

            def grp_body(g, carry):
                r0 = g * LANES

                def row_body(j, carry2):
                    r = r0 + j

                    def d_body(d, acc):
                        return acc + (u2[r, pl.ds(d * LANES, LANES)] *
                                      i2[r, pl.ds(d * LANES, LANES)])

                    acc = lax.fori_loop(
                        1, n_d, d_body,
                        u2[r, pl.ds(0, LANES)] * i2[r, pl.ds(0, LANES)])
                    part[pl.ds(j * LANES, LANES)] = acc
                    return carry2

                lax.fori_loop(0, LANES, row_body, 0)
                rowsel = lanes * LANES
                dots = plsc.load_gather(part, [rowsel])
                for l in range(1, LANES):
                    dots = dots + plsc.load_gather(part, [rowsel + l])
                outb[pl.ds(off + r0, LANES)] = dots
                return carry

            lax.fori_loop(0, sz // LANES, grp_body, 0)

        for cp in bias_cps:
            cp.wait()

        def bias_body(g, carry):
            r0 = g * LANES
            sl = pl.ds(r0, LANES)
            outb[sl] = outb[sl] + ub[sl] + ib[sl] + gbias
            return carry

        lax.fori_loop(0, b_per_w // LANES, bias_body, 0)

        pltpu.sync_copy(outb, out_hbm.at[pl.ds(base, b_per_w)])

    out = mf_kernel(user_ids, item_ids, user_table, item_table,
                    user_bias_table.T, item_bias_table.T, global_bias)
    return out.reshape(B, 1)
